# Initial kernel scaffold; baseline (speedup 1.0000x reference)
import jax
import jax.numpy as jnp
from jax.experimental import pallas as pl


def kernel(x, edge_index, batch, Wl0, bl0, Wr0, Wl1, bl1, Wr1, Wl2, bl2, Wr2, Wl3, bl3, Wr3, Wout, bout):
    raise NotImplementedError("write your pallas kernel here")


# speedup vs baseline: 2.1720x; 2.1720x over previous
import jax
import jax.numpy as jnp
from jax import lax
from jax.experimental import pallas as pl
from jax.experimental.pallas import tpu as pltpu
from jax.experimental.pallas import tpu_sc as plsc

_N = 10000
_E = 320000
_NG = 64
_NPAD = 10112
_ROWS_PER_TILE = 632
_CHUNK = 128
_EPAD = 327680
_BLK = 1000
_NBLK = 10


_CNT = 16384


def _sc_agg0_body(x_hbm, src_hbm, dst_hbm, zeros_hbm, agg_out, cnt_out,
                  src_v, dst_v, rows_v, cnt_v, sem, shared):
    c = lax.axis_index("c")
    s = lax.axis_index("s")
    row0 = s * _ROWS_PER_TILE
    pltpu.sync_copy(zeros_hbm.at[pl.ds(row0, _ROWS_PER_TILE)],
                    shared.at[pl.ds(row0, _ROWS_PER_TILE)])
    zeros16 = jnp.zeros((16,), jnp.float32)

    @pl.loop(0, _CNT // 16)
    def _(r):
        cnt_v[pl.ds(r * 16, 16)] = zeros16

    plsc.subcore_barrier()

    wid = c * 16 + s
    base = wid * (_EPAD // 32)
    ones16 = jnp.ones((16,), jnp.float32)

    @pl.loop(0, _EPAD // 32 // _CHUNK)
    def _(j):
        off = base + j * _CHUNK
        pltpu.sync_copy(src_hbm.at[pl.ds(off, _CHUNK)], src_v)
        pltpu.sync_copy(dst_hbm.at[pl.ds(off, _CHUNK)], dst_v)
        pltpu.async_copy(x_hbm.at[src_v], rows_v, sem).wait()
        pltpu.sync_copy(rows_v, shared.at[dst_v], add=True)
        for k in range(_CHUNK // 16):
            idx = dst_v[pl.ds(k * 16, 16)]
            plsc.addupdate_scatter(cnt_v, [idx], ones16)

    plsc.subcore_barrier()
    pltpu.sync_copy(shared.at[pl.ds(row0, _ROWS_PER_TILE)],
                    agg_out.at[c, pl.ds(row0, _ROWS_PER_TILE)])
    pltpu.sync_copy(cnt_v, cnt_out.at[wid])


def _make_sc_agg0():
    mesh = plsc.VectorSubcoreMesh(core_axis_name="c", subcore_axis_name="s",
                                  num_cores=2, num_subcores=16)
    return pl.kernel(
        _sc_agg0_body,
        out_type=[jax.ShapeDtypeStruct((2, _NPAD, 128), jnp.float32),
                  jax.ShapeDtypeStruct((32, _CNT), jnp.float32)],
        mesh=mesh,
        scratch_types=[
            pltpu.VMEM((_CHUNK,), jnp.int32),
            pltpu.VMEM((_CHUNK,), jnp.int32),
            pltpu.VMEM((_CHUNK, 128), jnp.float32),
            pltpu.VMEM((_CNT,), jnp.float32),
            pltpu.SemaphoreType.DMA,
            pltpu.VMEM_SHARED((_NPAD, 128), jnp.float32),
        ],
        compiler_params=pltpu.CompilerParams(needs_layout_passes=False),
        name="sc_agg0",
    )


def _sc_agg_body(h_hbm, src_hbm, dst_hbm, zeros_hbm, agg_out,
                 src_v, dst_v, rows_v, sem, shared):
    c = lax.axis_index("c")
    s = lax.axis_index("s")
    row0 = s * _ROWS_PER_TILE
    pltpu.sync_copy(zeros_hbm.at[pl.ds(row0, _ROWS_PER_TILE)],
                    shared.at[pl.ds(row0, _ROWS_PER_TILE)])
    plsc.subcore_barrier()

    base = s * (_EPAD // 16)

    @pl.loop(0, _EPAD // 16 // _CHUNK)
    def _(j):
        off = base + j * _CHUNK
        pltpu.sync_copy(src_hbm.at[pl.ds(off, _CHUNK)], src_v)
        pltpu.sync_copy(dst_hbm.at[pl.ds(off, _CHUNK)], dst_v)
        pltpu.async_copy(h_hbm.at[c].at[src_v], rows_v, sem).wait()
        pltpu.sync_copy(rows_v, shared.at[dst_v], add=True)

    plsc.subcore_barrier()
    pltpu.sync_copy(shared.at[pl.ds(row0, _ROWS_PER_TILE)],
                    agg_out.at[c, pl.ds(row0, _ROWS_PER_TILE)])


def _make_sc_agg():
    mesh = plsc.VectorSubcoreMesh(core_axis_name="c", subcore_axis_name="s",
                                  num_cores=2, num_subcores=16)
    return pl.kernel(
        _sc_agg_body,
        out_type=jax.ShapeDtypeStruct((2, _NPAD, 128), jnp.float32),
        mesh=mesh,
        scratch_types=[
            pltpu.VMEM((_CHUNK,), jnp.int32),
            pltpu.VMEM((_CHUNK,), jnp.int32),
            pltpu.VMEM((_CHUNK, 128), jnp.float32),
            pltpu.SemaphoreType.DMA,
            pltpu.VMEM_SHARED((_NPAD, 128), jnp.float32),
        ],
        name="sc_agg",
    )


def _split(v, pad):
    lo = v[:, :128]
    hi = jnp.concatenate([v[:, 128:], pad], axis=1)
    return lo, hi


def _tc_layer0_body(aref, cref, xref, wlref, blref, wrref, oref, invcref):
    cnt = jnp.sum(cref[...], axis=0)
    invc = 1.0 / jnp.maximum(cnt, 1.0)
    mean = (aref[0] + aref[1]) * invc
    acc = (jnp.dot(mean, wlref[...], preferred_element_type=jnp.float32)
           + jnp.dot(xref[...], wrref[...], preferred_element_type=jnp.float32)
           + blref[...])
    acc = jnp.maximum(acc, 0.0)
    lo, hi = _split(acc, jnp.zeros((_BLK, 64), jnp.float32))
    oref[0] = lo
    oref[1] = hi
    invcref[...] = invc


def _tc_layer_body(aref, href, invcref, wlref, blref, wrref, oref):
    invc = invcref[...]
    mean = jnp.concatenate([aref[0], aref[1][:, :64]], axis=1) * invc
    h = jnp.concatenate([href[0], href[1][:, :64]], axis=1)
    acc = (jnp.dot(mean, wlref[...], preferred_element_type=jnp.float32)
           + jnp.dot(h, wrref[...], preferred_element_type=jnp.float32)
           + blref[...])
    acc = jnp.maximum(acc, 0.0)
    lo, hi = _split(acc, jnp.zeros((_BLK, 64), jnp.float32))
    oref[0] = lo
    oref[1] = hi


def _tc_pool_body(href, bref, wref, boutref, outref, pooledref, psum, pcnt):
    i = pl.program_id(0)

    @pl.when(i == 0)
    def _():
        psum[...] = jnp.zeros_like(psum)
        pcnt[...] = jnp.zeros_like(pcnt)

    h = jnp.concatenate([href[0], href[1][:, :64]], axis=1)
    b = bref[0, 0]
    gids = lax.broadcasted_iota(jnp.int32, (_BLK, _NG), 1).astype(jnp.float32)
    onehot = jnp.where(b[:, None] == gids, 1.0, 0.0)
    psum[...] += lax.dot_general(onehot, h, (((0,), (0,)), ((), ())),
                                 preferred_element_type=jnp.float32)
    pcnt[...] += jnp.sum(onehot, axis=0, keepdims=True)

    @pl.when(i == _NBLK - 1)
    def _():
        pooled = psum[...] / jnp.maximum(pcnt[...], 1.0).T
        pooledref[...] = pooled
        outref[...] = (jnp.dot(pooled, wref[...],
                               preferred_element_type=jnp.float32)
                       + boutref[...])


def kernel(x, edge_index, batch, Wl0, bl0, Wr0, Wl1, bl1, Wr1,
           Wl2, bl2, Wr2, Wl3, bl3, Wr3, Wout, bout):
    f32 = jnp.float32
    src = jnp.concatenate([edge_index[0], jnp.zeros((_EPAD - _E,), jnp.int32)])
    dst = jnp.concatenate([edge_index[1],
                           jnp.full((_EPAD - _E,), _N, jnp.int32)])
    zeros128 = jnp.zeros((_NPAD, 128), f32)

    agg0, cntp = _make_sc_agg0()(x, src, dst, zeros128)

    full = lambda d: pl.BlockSpec(d, lambda i: tuple(0 for _ in d))
    rowb = lambda d: pl.BlockSpec((_BLK, d), lambda i: (i, 0))
    splitb = pl.BlockSpec((2, _BLK, 128), lambda i: (0, i, 0))
    cntb = pl.BlockSpec((32, _BLK, 1), lambda i: (0, i, 0))

    h1, invc = pl.pallas_call(
        _tc_layer0_body,
        grid=(_NBLK,),
        in_specs=[splitb, cntb, rowb(128), full((128, 192)), full((1, 192)),
                  full((128, 192))],
        out_specs=[pl.BlockSpec((2, _BLK, 128), lambda i: (0, i, 0)),
                   rowb(1)],
        out_shape=[jax.ShapeDtypeStruct((2, _N, 128), f32),
                   jax.ShapeDtypeStruct((_N, 1), f32)],
        name="tc_layer0",
    )(agg0, cntp.reshape(32, _CNT, 1), x, Wl0, bl0.reshape(1, 192), Wr0)

    sc_agg = _make_sc_agg()
    tc_layer = pl.pallas_call(
        _tc_layer_body,
        grid=(_NBLK,),
        in_specs=[splitb, splitb, rowb(1), full((192, 192)),
                  full((1, 192)), full((192, 192))],
        out_specs=pl.BlockSpec((2, _BLK, 128), lambda i: (0, i, 0)),
        out_shape=jax.ShapeDtypeStruct((2, _N, 128), f32),
        name="tc_layer",
    )

    h = h1
    for (Wl, bl, Wr) in ((Wl1, bl1, Wr1), (Wl2, bl2, Wr2), (Wl3, bl3, Wr3)):
        agg = sc_agg(h, src, dst, zeros128)
        h = tc_layer(agg, h, invc, Wl, bl.reshape(1, 192), Wr)

    batchf = batch.astype(f32).reshape(_NBLK, 1, _BLK)
    out, pooled = pl.pallas_call(
        _tc_pool_body,
        grid=(_NBLK,),
        in_specs=[splitb, pl.BlockSpec((1, 1, _BLK), lambda i: (i, 0, 0)),
                  full((192, 1)), full((1, 1))],
        out_specs=[full((_NG, 1)), full((_NG, 192))],
        out_shape=[jax.ShapeDtypeStruct((_NG, 1), f32),
                   jax.ShapeDtypeStruct((_NG, 192), f32)],
        scratch_shapes=[pltpu.VMEM((_NG, 192), f32),
                        pltpu.VMEM((1, _NG), f32)],
        name="tc_pool",
    )(h, batchf, Wout, bout.reshape(1, 1))

    return (out, pooled)


# speedup vs baseline: 2.9039x; 1.3370x over previous
import jax
import jax.numpy as jnp
from jax import lax
from jax.experimental import pallas as pl
from jax.experimental.pallas import tpu as pltpu
from jax.experimental.pallas import tpu_sc as plsc

_N = 10000
_E = 320000
_NG = 64
_NPAD = 10112
_ROWS_PER_TILE = 632
_CHUNK = 128
_EPAD = 327680
_BLK = 1000
_NBLK = 10


_CNT = 10112


def _edge_pipeline(plane_hbm, src_hbm, dst_hbm, si, di, rows, gsem, ssem,
                   xsem, ysem, shared, base, nc, per_chunk=None):
    pltpu.sync_copy(src_hbm.at[base], si[0])
    pltpu.sync_copy(dst_hbm.at[base], di[0])
    pltpu.async_copy(plane_hbm.at[si[0]], rows[0], gsem[0])
    pltpu.async_copy(src_hbm.at[base + 1], si[1], xsem[1])
    pltpu.async_copy(dst_hbm.at[base + 1], di[1], ysem[1])

    @pl.loop(0, nc // 2)
    def _(jj):
        for b in range(2):
            j = jj * 2 + b
            o = 1 - b
            pltpu.make_async_copy(plane_hbm.at[si[b]], rows[b],
                                  gsem[b]).wait()

            @pl.when(j + 2 < nc)
            def _():
                pltpu.async_copy(src_hbm.at[base + j + 2], si[b], xsem[b])

            @pl.when(j + 1 < nc)
            def _():
                @pl.when(j >= 1)
                def _():
                    pltpu.make_async_copy(rows[o], shared.at[di[o]],
                                          ssem[o]).wait()
                    pltpu.async_copy(dst_hbm.at[base + j + 1], di[o],
                                     ysem[o])
                pltpu.make_async_copy(src_hbm.at[base + j + 1], si[o],
                                      xsem[o]).wait()
                pltpu.async_copy(plane_hbm.at[si[o]], rows[o], gsem[o])

            @pl.when(j >= 1)
            def _():
                pltpu.make_async_copy(dst_hbm.at[base + j], di[b],
                                      ysem[b]).wait()
            pltpu.async_copy(rows[b], shared.at[di[b]], ssem[b], add=True)
            if per_chunk is not None:
                per_chunk(b)

    pltpu.make_async_copy(rows[0], shared.at[di[0]], ssem[0]).wait()
    pltpu.make_async_copy(rows[1], shared.at[di[1]], ssem[1]).wait()


def _sc_agg0_body(x_hbm, src_hbm, dst_hbm, zeros_hbm, agg_out, cnt_out,
                  si0, si1, di0, di1, rows0, rows1, cnt_v,
                  gsem0, gsem1, ssem0, ssem1, xsem0, xsem1, ysem0, ysem1,
                  shared):
    c = lax.axis_index("c")
    s = lax.axis_index("s")
    row0 = s * _ROWS_PER_TILE
    pltpu.sync_copy(zeros_hbm.at[pl.ds(row0, _ROWS_PER_TILE)],
                    shared.at[pl.ds(row0, _ROWS_PER_TILE)])
    zeros16 = jnp.zeros((16,), jnp.float32)

    @pl.loop(0, _CNT // 16)
    def _(r):
        cnt_v[pl.ds(r * 16, 16)] = zeros16

    plsc.subcore_barrier()

    wid = c * 16 + s
    nc = _EPAD // 32 // _CHUNK
    ones16 = jnp.ones((16,), jnp.float32)
    di = [di0, di1]

    def count(b):
        for k in range(_CHUNK // 16):
            idx = di[b][pl.ds(k * 16, 16)]
            plsc.addupdate_scatter(cnt_v, [idx], ones16)

    _edge_pipeline(x_hbm, src_hbm, dst_hbm, [si0, si1], di, [rows0, rows1],
                   [gsem0, gsem1], [ssem0, ssem1], [xsem0, xsem1],
                   [ysem0, ysem1], shared, wid * nc, nc, per_chunk=count)

    plsc.subcore_barrier()
    pltpu.sync_copy(shared.at[pl.ds(row0, _ROWS_PER_TILE)],
                    agg_out.at[c, pl.ds(row0, _ROWS_PER_TILE)])
    pltpu.sync_copy(cnt_v, cnt_out.at[wid])


def _make_sc_agg0():
    mesh = plsc.VectorSubcoreMesh(core_axis_name="c", subcore_axis_name="s",
                                  num_cores=2, num_subcores=16)
    return pl.kernel(
        _sc_agg0_body,
        out_type=[jax.ShapeDtypeStruct((2, _NPAD, 128), jnp.float32),
                  jax.ShapeDtypeStruct((32, _CNT), jnp.float32)],
        mesh=mesh,
        scratch_types=[
            pltpu.VMEM((_CHUNK,), jnp.int32),
            pltpu.VMEM((_CHUNK,), jnp.int32),
            pltpu.VMEM((_CHUNK,), jnp.int32),
            pltpu.VMEM((_CHUNK,), jnp.int32),
            pltpu.VMEM((_CHUNK, 128), jnp.float32),
            pltpu.VMEM((_CHUNK, 128), jnp.float32),
            pltpu.VMEM((_CNT,), jnp.float32),
            pltpu.SemaphoreType.DMA,
            pltpu.SemaphoreType.DMA,
            pltpu.SemaphoreType.DMA,
            pltpu.SemaphoreType.DMA,
            pltpu.SemaphoreType.DMA,
            pltpu.SemaphoreType.DMA,
            pltpu.SemaphoreType.DMA,
            pltpu.SemaphoreType.DMA,
            pltpu.VMEM_SHARED((_NPAD, 128), jnp.float32),
        ],
        compiler_params=pltpu.CompilerParams(needs_layout_passes=False),
        name="sc_agg0",
    )


def _sc_agg_body(h_hbm, src_hbm, dst_hbm, zeros_hbm, agg_out,
                 si0, si1, di0, di1, rows0, rows1,
                 gsem0, gsem1, ssem0, ssem1, xsem0, xsem1, ysem0, ysem1,
                 shared):
    c = lax.axis_index("c")
    s = lax.axis_index("s")
    row0 = s * _ROWS_PER_TILE
    pltpu.sync_copy(zeros_hbm.at[pl.ds(row0, _ROWS_PER_TILE)],
                    shared.at[pl.ds(row0, _ROWS_PER_TILE)])
    plsc.subcore_barrier()

    nc = _EPAD // 16 // _CHUNK
    _edge_pipeline(h_hbm.at[c], src_hbm, dst_hbm, [si0, si1], [di0, di1],
                   [rows0, rows1], [gsem0, gsem1], [ssem0, ssem1],
                   [xsem0, xsem1], [ysem0, ysem1], shared, s * nc, nc)

    plsc.subcore_barrier()
    pltpu.sync_copy(shared.at[pl.ds(row0, _ROWS_PER_TILE)],
                    agg_out.at[c, pl.ds(row0, _ROWS_PER_TILE)])


def _make_sc_agg():
    mesh = plsc.VectorSubcoreMesh(core_axis_name="c", subcore_axis_name="s",
                                  num_cores=2, num_subcores=16)
    return pl.kernel(
        _sc_agg_body,
        out_type=jax.ShapeDtypeStruct((2, _NPAD, 128), jnp.float32),
        mesh=mesh,
        scratch_types=[
            pltpu.VMEM((_CHUNK,), jnp.int32),
            pltpu.VMEM((_CHUNK,), jnp.int32),
            pltpu.VMEM((_CHUNK,), jnp.int32),
            pltpu.VMEM((_CHUNK,), jnp.int32),
            pltpu.VMEM((_CHUNK, 128), jnp.float32),
            pltpu.VMEM((_CHUNK, 128), jnp.float32),
            pltpu.SemaphoreType.DMA,
            pltpu.SemaphoreType.DMA,
            pltpu.SemaphoreType.DMA,
            pltpu.SemaphoreType.DMA,
            pltpu.SemaphoreType.DMA,
            pltpu.SemaphoreType.DMA,
            pltpu.SemaphoreType.DMA,
            pltpu.SemaphoreType.DMA,
            pltpu.VMEM_SHARED((_NPAD, 128), jnp.float32),
        ],
        compiler_params=pltpu.CompilerParams(needs_layout_passes=False),
        name="sc_agg",
    )


def _split(v, pad):
    lo = v[:, :128]
    hi = jnp.concatenate([v[:, 128:], pad], axis=1)
    return lo, hi


def _tc_layer0_body(aref, cref, xref, wlref, blref, wrref, oref, invcref):
    cnt = jnp.sum(cref[...], axis=0)
    invc = 1.0 / jnp.maximum(cnt, 1.0)
    mean = (aref[0] + aref[1]) * invc
    acc = (jnp.dot(mean, wlref[...], preferred_element_type=jnp.float32)
           + jnp.dot(xref[...], wrref[...], preferred_element_type=jnp.float32)
           + blref[...])
    acc = jnp.maximum(acc, 0.0)
    lo, hi = _split(acc, jnp.zeros((_BLK, 64), jnp.float32))
    oref[0] = lo
    oref[1] = hi
    invcref[...] = invc


def _tc_layer_body(aref, href, invcref, wlref, blref, wrref, oref):
    invc = invcref[...]
    mean = jnp.concatenate([aref[0], aref[1][:, :64]], axis=1) * invc
    h = jnp.concatenate([href[0], href[1][:, :64]], axis=1)
    acc = (jnp.dot(mean, wlref[...], preferred_element_type=jnp.float32)
           + jnp.dot(h, wrref[...], preferred_element_type=jnp.float32)
           + blref[...])
    acc = jnp.maximum(acc, 0.0)
    lo, hi = _split(acc, jnp.zeros((_BLK, 64), jnp.float32))
    oref[0] = lo
    oref[1] = hi


def _tc_pool_body(href, bref, wref, boutref, outref, pooledref, psum, pcnt):
    i = pl.program_id(0)

    @pl.when(i == 0)
    def _():
        psum[...] = jnp.zeros_like(psum)
        pcnt[...] = jnp.zeros_like(pcnt)

    h = jnp.concatenate([href[0], href[1][:, :64]], axis=1)
    b = bref[0, 0]
    gids = lax.broadcasted_iota(jnp.int32, (_BLK, _NG), 1).astype(jnp.float32)
    onehot = jnp.where(b[:, None] == gids, 1.0, 0.0)
    psum[...] += lax.dot_general(onehot, h, (((0,), (0,)), ((), ())),
                                 preferred_element_type=jnp.float32)
    pcnt[...] += jnp.sum(onehot, axis=0, keepdims=True)

    @pl.when(i == _NBLK - 1)
    def _():
        pooled = psum[...] / jnp.maximum(pcnt[...], 1.0).T
        pooledref[...] = pooled
        outref[...] = (jnp.dot(pooled, wref[...],
                               preferred_element_type=jnp.float32)
                       + boutref[...])


def kernel(x, edge_index, batch, Wl0, bl0, Wr0, Wl1, bl1, Wr1,
           Wl2, bl2, Wr2, Wl3, bl3, Wr3, Wout, bout):
    f32 = jnp.float32
    src = jnp.concatenate(
        [edge_index[0], jnp.zeros((_EPAD - _E,), jnp.int32)]
    ).reshape(_EPAD // _CHUNK, _CHUNK)
    dst = jnp.concatenate(
        [edge_index[1], jnp.full((_EPAD - _E,), _N, jnp.int32)]
    ).reshape(_EPAD // _CHUNK, _CHUNK)
    zeros128 = jnp.zeros((_NPAD, 128), f32)

    agg0, cntp = _make_sc_agg0()(x, src, dst, zeros128)

    full = lambda d: pl.BlockSpec(d, lambda i: tuple(0 for _ in d))
    rowb = lambda d: pl.BlockSpec((_BLK, d), lambda i: (i, 0))
    splitb = pl.BlockSpec((2, _BLK, 128), lambda i: (0, i, 0))
    cntb = pl.BlockSpec((32, _BLK, 1), lambda i: (0, i, 0))

    h1, invc = pl.pallas_call(
        _tc_layer0_body,
        grid=(_NBLK,),
        in_specs=[splitb, cntb, rowb(128), full((128, 192)), full((1, 192)),
                  full((128, 192))],
        out_specs=[pl.BlockSpec((2, _BLK, 128), lambda i: (0, i, 0)),
                   rowb(1)],
        out_shape=[jax.ShapeDtypeStruct((2, _N, 128), f32),
                   jax.ShapeDtypeStruct((_N, 1), f32)],
        name="tc_layer0",
    )(agg0, cntp.reshape(32, _CNT, 1), x, Wl0, bl0.reshape(1, 192), Wr0)

    sc_agg = _make_sc_agg()
    tc_layer = pl.pallas_call(
        _tc_layer_body,
        grid=(_NBLK,),
        in_specs=[splitb, splitb, rowb(1), full((192, 192)),
                  full((1, 192)), full((192, 192))],
        out_specs=pl.BlockSpec((2, _BLK, 128), lambda i: (0, i, 0)),
        out_shape=jax.ShapeDtypeStruct((2, _N, 128), f32),
        name="tc_layer",
    )

    h = h1
    for (Wl, bl, Wr) in ((Wl1, bl1, Wr1), (Wl2, bl2, Wr2), (Wl3, bl3, Wr3)):
        agg = sc_agg(h, src, dst, zeros128)
        h = tc_layer(agg, h, invc, Wl, bl.reshape(1, 192), Wr)

    batchf = batch.astype(f32).reshape(_NBLK, 1, _BLK)
    out, pooled = pl.pallas_call(
        _tc_pool_body,
        grid=(_NBLK,),
        in_specs=[splitb, pl.BlockSpec((1, 1, _BLK), lambda i: (i, 0, 0)),
                  full((192, 1)), full((1, 1))],
        out_specs=[full((_NG, 1)), full((_NG, 192))],
        out_shape=[jax.ShapeDtypeStruct((_NG, 1), f32),
                   jax.ShapeDtypeStruct((_NG, 192), f32)],
        scratch_shapes=[pltpu.VMEM((_NG, 192), f32),
                        pltpu.VMEM((1, _NG), f32)],
        name="tc_pool",
    )(h, batchf, Wout, bout.reshape(1, 1))

    return (out, pooled)


# speedup vs baseline: 2.9361x; 1.0111x over previous
import jax
import jax.numpy as jnp
from jax import lax
from jax.experimental import pallas as pl
from jax.experimental.pallas import tpu as pltpu
from jax.experimental.pallas import tpu_sc as plsc

_N = 10000
_E = 320000
_NG = 64
_NPAD = 10112
_ROWS_PER_TILE = 632
_CHUNK = 64
_R = 4
_D = 3
_EPAD = 327680
_BLK = 1000
_NBLK = 10


_CNT = 10112


def _edge_pipeline(plane_hbm, src_hbm, dst_hbm, si, di, rows, gsem, ssem,
                   xsem, ysem, shared, base, nc, per_chunk=None):
    for k in range(_D):
        pltpu.sync_copy(src_hbm.at[base + k], si[k])
        pltpu.sync_copy(dst_hbm.at[base + k], di[k])
        pltpu.async_copy(plane_hbm.at[si[k]], rows[k], gsem[k])
    for k in range(_D, _R):
        pltpu.async_copy(src_hbm.at[base + k], si[k], xsem[k])
        pltpu.async_copy(dst_hbm.at[base + k], di[k], ysem[k])

    @pl.loop(0, nc // _R)
    def _(jj):
        for b in range(_R):
            j = jj * _R + b
            g = (b + _D) % _R
            p = (b + _R - 1) % _R
            pltpu.make_async_copy(plane_hbm.at[si[b]], rows[b],
                                  gsem[b]).wait()

            @pl.when(j >= 1)
            def _():
                pltpu.make_async_copy(rows[p], shared.at[di[p]],
                                      ssem[p]).wait()

            @pl.when(j + _D < nc)
            def _():
                @pl.when(j + _D - _R >= 0)
                def _():
                    pltpu.async_copy(dst_hbm.at[base + j + _D], di[g],
                                     ysem[g])
                pltpu.make_async_copy(src_hbm.at[base + j + _D], si[g],
                                      xsem[g]).wait()
                pltpu.async_copy(plane_hbm.at[si[g]], rows[g], gsem[g])

            @pl.when(j + _R < nc)
            def _():
                pltpu.async_copy(src_hbm.at[base + j + _R], si[b], xsem[b])

            @pl.when(j >= _D)
            def _():
                pltpu.make_async_copy(dst_hbm.at[base + j], di[b],
                                      ysem[b]).wait()
            pltpu.async_copy(rows[b], shared.at[di[b]], ssem[b], add=True)
            if per_chunk is not None:
                per_chunk(b)

    pltpu.make_async_copy(rows[(nc - 1) % _R], shared.at[di[(nc - 1) % _R]],
                          ssem[(nc - 1) % _R]).wait()


def _sc_agg0_body(x_hbm, src_hbm, dst_hbm, zeros_hbm, agg_out, cnt_out,
                  *scr):
    si, di, rows = list(scr[0:_R]), list(scr[_R:2 * _R]), \
        list(scr[2 * _R:3 * _R])
    cnt_v = scr[3 * _R]
    gsem = list(scr[3 * _R + 1:4 * _R + 1])
    ssem = list(scr[4 * _R + 1:5 * _R + 1])
    xsem = list(scr[5 * _R + 1:6 * _R + 1])
    ysem = list(scr[6 * _R + 1:7 * _R + 1])
    shared = scr[7 * _R + 1]
    c = lax.axis_index("c")
    s = lax.axis_index("s")
    row0 = s * _ROWS_PER_TILE
    pltpu.sync_copy(zeros_hbm.at[pl.ds(row0, _ROWS_PER_TILE)],
                    shared.at[pl.ds(row0, _ROWS_PER_TILE)])
    zeros16 = jnp.zeros((16,), jnp.float32)

    @pl.loop(0, _CNT // 16)
    def _(r):
        cnt_v[pl.ds(r * 16, 16)] = zeros16

    plsc.subcore_barrier()

    wid = c * 16 + s
    nc = _EPAD // 32 // _CHUNK
    ones16 = jnp.ones((16,), jnp.float32)

    def count(b):
        for k in range(_CHUNK // 16):
            idx = di[b][pl.ds(k * 16, 16)]
            plsc.addupdate_scatter(cnt_v, [idx], ones16)

    _edge_pipeline(x_hbm, src_hbm, dst_hbm, si, di, rows, gsem, ssem,
                   xsem, ysem, shared, wid * nc, nc, per_chunk=count)

    plsc.subcore_barrier()
    pltpu.sync_copy(shared.at[pl.ds(row0, _ROWS_PER_TILE)],
                    agg_out.at[c, pl.ds(row0, _ROWS_PER_TILE)])
    pltpu.sync_copy(cnt_v, cnt_out.at[wid])


def _make_sc_agg0():
    mesh = plsc.VectorSubcoreMesh(core_axis_name="c", subcore_axis_name="s",
                                  num_cores=2, num_subcores=16)
    return pl.kernel(
        _sc_agg0_body,
        out_type=[jax.ShapeDtypeStruct((2, _NPAD, 128), jnp.float32),
                  jax.ShapeDtypeStruct((32, _CNT), jnp.float32)],
        mesh=mesh,
        scratch_types=(
            [pltpu.VMEM((_CHUNK,), jnp.int32)] * (2 * _R)
            + [pltpu.VMEM((_CHUNK, 128), jnp.float32)] * _R
            + [pltpu.VMEM((_CNT,), jnp.float32)]
            + [pltpu.SemaphoreType.DMA] * (4 * _R)
            + [pltpu.VMEM_SHARED((_NPAD, 128), jnp.float32)]
        ),
        compiler_params=pltpu.CompilerParams(needs_layout_passes=False),
        name="sc_agg0",
    )


def _sc_agg_body(h_hbm, src_hbm, dst_hbm, zeros_hbm, agg_out, *scr):
    si, di, rows = list(scr[0:_R]), list(scr[_R:2 * _R]), \
        list(scr[2 * _R:3 * _R])
    gsem = list(scr[3 * _R:4 * _R])
    ssem = list(scr[4 * _R:5 * _R])
    xsem = list(scr[5 * _R:6 * _R])
    ysem = list(scr[6 * _R:7 * _R])
    shared = scr[7 * _R]
    c = lax.axis_index("c")
    s = lax.axis_index("s")
    row0 = s * _ROWS_PER_TILE
    pltpu.sync_copy(zeros_hbm.at[pl.ds(row0, _ROWS_PER_TILE)],
                    shared.at[pl.ds(row0, _ROWS_PER_TILE)])
    plsc.subcore_barrier()

    nc = _EPAD // 16 // _CHUNK
    _edge_pipeline(h_hbm.at[c], src_hbm, dst_hbm, si, di, rows, gsem, ssem,
                   xsem, ysem, shared, s * nc, nc)

    plsc.subcore_barrier()
    pltpu.sync_copy(shared.at[pl.ds(row0, _ROWS_PER_TILE)],
                    agg_out.at[c, pl.ds(row0, _ROWS_PER_TILE)])


def _make_sc_agg():
    mesh = plsc.VectorSubcoreMesh(core_axis_name="c", subcore_axis_name="s",
                                  num_cores=2, num_subcores=16)
    return pl.kernel(
        _sc_agg_body,
        out_type=jax.ShapeDtypeStruct((2, _NPAD, 128), jnp.float32),
        mesh=mesh,
        scratch_types=(
            [pltpu.VMEM((_CHUNK,), jnp.int32)] * (2 * _R)
            + [pltpu.VMEM((_CHUNK, 128), jnp.float32)] * _R
            + [pltpu.SemaphoreType.DMA] * (4 * _R)
            + [pltpu.VMEM_SHARED((_NPAD, 128), jnp.float32)]
        ),
        compiler_params=pltpu.CompilerParams(needs_layout_passes=False),
        name="sc_agg",
    )


def _split(v, pad):
    lo = v[:, :128]
    hi = jnp.concatenate([v[:, 128:], pad], axis=1)
    return lo, hi


def _tc_layer0_body(aref, cref, xref, wlref, blref, wrref, oref, invcref):
    cnt = jnp.sum(cref[...], axis=0)
    invc = 1.0 / jnp.maximum(cnt, 1.0)
    mean = (aref[0] + aref[1]) * invc
    acc = (jnp.dot(mean, wlref[...], preferred_element_type=jnp.float32)
           + jnp.dot(xref[...], wrref[...], preferred_element_type=jnp.float32)
           + blref[...])
    acc = jnp.maximum(acc, 0.0)
    lo, hi = _split(acc, jnp.zeros((_BLK, 64), jnp.float32))
    oref[0] = lo
    oref[1] = hi
    invcref[...] = invc


def _tc_layer_body(aref, href, invcref, wlref, blref, wrref, oref):
    invc = invcref[...]
    mean = jnp.concatenate([aref[0], aref[1][:, :64]], axis=1) * invc
    h = jnp.concatenate([href[0], href[1][:, :64]], axis=1)
    acc = (jnp.dot(mean, wlref[...], preferred_element_type=jnp.float32)
           + jnp.dot(h, wrref[...], preferred_element_type=jnp.float32)
           + blref[...])
    acc = jnp.maximum(acc, 0.0)
    lo, hi = _split(acc, jnp.zeros((_BLK, 64), jnp.float32))
    oref[0] = lo
    oref[1] = hi


def _tc_pool_body(href, bref, wref, boutref, outref, pooledref, psum, pcnt):
    i = pl.program_id(0)

    @pl.when(i == 0)
    def _():
        psum[...] = jnp.zeros_like(psum)
        pcnt[...] = jnp.zeros_like(pcnt)

    h = jnp.concatenate([href[0], href[1][:, :64]], axis=1)
    b = bref[0, 0]
    gids = lax.broadcasted_iota(jnp.int32, (_BLK, _NG), 1).astype(jnp.float32)
    onehot = jnp.where(b[:, None] == gids, 1.0, 0.0)
    psum[...] += lax.dot_general(onehot, h, (((0,), (0,)), ((), ())),
                                 preferred_element_type=jnp.float32)
    pcnt[...] += jnp.sum(onehot, axis=0, keepdims=True)

    @pl.when(i == _NBLK - 1)
    def _():
        pooled = psum[...] / jnp.maximum(pcnt[...], 1.0).T
        pooledref[...] = pooled
        outref[...] = (jnp.dot(pooled, wref[...],
                               preferred_element_type=jnp.float32)
                       + boutref[...])


def kernel(x, edge_index, batch, Wl0, bl0, Wr0, Wl1, bl1, Wr1,
           Wl2, bl2, Wr2, Wl3, bl3, Wr3, Wout, bout):
    f32 = jnp.float32
    src = jnp.concatenate(
        [edge_index[0], jnp.zeros((_EPAD - _E,), jnp.int32)]
    ).reshape(_EPAD // _CHUNK, _CHUNK)
    dst = jnp.concatenate(
        [edge_index[1], jnp.full((_EPAD - _E,), _N, jnp.int32)]
    ).reshape(_EPAD // _CHUNK, _CHUNK)
    zeros128 = jnp.zeros((_NPAD, 128), f32)

    agg0, cntp = _make_sc_agg0()(x, src, dst, zeros128)

    full = lambda d: pl.BlockSpec(d, lambda i: tuple(0 for _ in d))
    rowb = lambda d: pl.BlockSpec((_BLK, d), lambda i: (i, 0))
    splitb = pl.BlockSpec((2, _BLK, 128), lambda i: (0, i, 0))
    cntb = pl.BlockSpec((32, _BLK, 1), lambda i: (0, i, 0))

    h1, invc = pl.pallas_call(
        _tc_layer0_body,
        grid=(_NBLK,),
        in_specs=[splitb, cntb, rowb(128), full((128, 192)), full((1, 192)),
                  full((128, 192))],
        out_specs=[pl.BlockSpec((2, _BLK, 128), lambda i: (0, i, 0)),
                   rowb(1)],
        out_shape=[jax.ShapeDtypeStruct((2, _N, 128), f32),
                   jax.ShapeDtypeStruct((_N, 1), f32)],
        name="tc_layer0",
    )(agg0, cntp.reshape(32, _CNT, 1), x, Wl0, bl0.reshape(1, 192), Wr0)

    sc_agg = _make_sc_agg()
    tc_layer = pl.pallas_call(
        _tc_layer_body,
        grid=(_NBLK,),
        in_specs=[splitb, splitb, rowb(1), full((192, 192)),
                  full((1, 192)), full((192, 192))],
        out_specs=pl.BlockSpec((2, _BLK, 128), lambda i: (0, i, 0)),
        out_shape=jax.ShapeDtypeStruct((2, _N, 128), f32),
        name="tc_layer",
    )

    h = h1
    for (Wl, bl, Wr) in ((Wl1, bl1, Wr1), (Wl2, bl2, Wr2), (Wl3, bl3, Wr3)):
        agg = sc_agg(h, src, dst, zeros128)
        h = tc_layer(agg, h, invc, Wl, bl.reshape(1, 192), Wr)

    batchf = batch.astype(f32).reshape(_NBLK, 1, _BLK)
    out, pooled = pl.pallas_call(
        _tc_pool_body,
        grid=(_NBLK,),
        in_specs=[splitb, pl.BlockSpec((1, 1, _BLK), lambda i: (i, 0, 0)),
                  full((192, 1)), full((1, 1))],
        out_specs=[full((_NG, 1)), full((_NG, 192))],
        out_shape=[jax.ShapeDtypeStruct((_NG, 1), f32),
                   jax.ShapeDtypeStruct((_NG, 192), f32)],
        scratch_shapes=[pltpu.VMEM((_NG, 192), f32),
                        pltpu.VMEM((1, _NG), f32)],
        name="tc_pool",
    )(h, batchf, Wout, bout.reshape(1, 1))

    return (out, pooled)
